# baseline (speedup 1.0000x reference)
import functools
from math import ceil

import numpy as np
import jax
import jax.numpy as jnp
from jax.experimental import pallas as pl
from jax.experimental.pallas import tpu as pltpu

_SEQ_LEN = 336
_D_MODEL = 512
_NUM_VARIATES = 11
_BATCH = 16


def _compute_patch_sizes(seq_len):
    freqs = np.fft.rfftfreq(seq_len)[1:]
    periods = 1.0 / freqs
    return np.unique(np.floor(periods).astype(np.int64))[::-1].copy()


_PATCH_SIZES = [int(p) for p in _compute_patch_sizes(_SEQ_LEN)]
_NS = [ceil(_SEQ_LEN / p) for p in _PATCH_SIZES]


def _sin_pe_np(L, d):
    pos = np.arange(L, dtype=np.float64)[:, None]
    div = np.exp(np.arange(0, d, 2, dtype=np.float64) * (-np.log(10000.0) / d))
    pe = np.zeros((L, d), dtype=np.float64)
    pe[:, 0::2] = np.sin(pos * div)
    pe[:, 1::2] = np.cos(pos * div)
    return pe


def _pe_rows_np(C, N, d_model):
    dh = d_model // 2
    pe = np.zeros((C, N, d_model), dtype=np.float32)
    pe[:, :, :dh] = _sin_pe_np(C, dh)[:, None, :]
    pe[:, :, dh:] = _sin_pe_np(N, d_model - dh)[None, :, :]
    return pe.reshape(C * N, d_model)


_PE_ROWS = {n: jnp.asarray(_pe_rows_np(_NUM_VARIATES, n, _D_MODEL))
            for n in sorted(set(_NS))}


def _expert_body(order_ref, x_ref, w_ref, pe_ref, o_ref):
    xi = x_ref[0]
    w = w_ref[...]
    acc = jax.lax.dot_general(
        xi, w, (((1,), (1,)), ((), ())),
        preferred_element_type=jnp.float32)
    o_ref[0] = acc + pe_ref[...]


@functools.partial(jax.jit, static_argnums=(0, 1))
def _expert_call(p, n, x_rows, w, pe, order):
    rows = _NUM_VARIATES * n
    grid_spec = pltpu.PrefetchScalarGridSpec(
        num_scalar_prefetch=1,
        grid=(_BATCH,),
        in_specs=[
            pl.BlockSpec((1, rows, p), lambda b, order: (order[b], 0, 0)),
            pl.BlockSpec((_D_MODEL, p), lambda b, order: (0, 0)),
            pl.BlockSpec((rows, _D_MODEL), lambda b, order: (0, 0)),
        ],
        out_specs=pl.BlockSpec((1, rows, _D_MODEL), lambda b, order: (b, 0, 0)),
    )
    out = pl.pallas_call(
        _expert_body,
        grid_spec=grid_spec,
        out_shape=jax.ShapeDtypeStruct((_BATCH, rows, _D_MODEL), jnp.float32),
    )(order, x_rows, w, pe)
    return out


def kernel(x, gates, Ws):
    batch_ar = jnp.arange(_BATCH, dtype=jnp.int32)[:, None]
    keys = jnp.where(gates != 0, jnp.int32(0), jnp.int32(1)) * (_BATCH + 1) + batch_ar
    orders = jnp.argsort(keys, axis=0, stable=True).astype(jnp.int32)
    orders = orders.T

    xt = jnp.swapaxes(x, 1, 2)
    xt_pad = jnp.concatenate(
        [xt, jnp.broadcast_to(xt[:, :, -1:], (_BATCH, _NUM_VARIATES, _SEQ_LEN))],
        axis=-1)

    outs = []
    for i, p in enumerate(_PATCH_SIZES):
        n = _NS[i]
        x_rows = xt_pad[:, :, : n * p].reshape(_BATCH, _NUM_VARIATES * n, p)
        out = _expert_call(p, n, x_rows, Ws[i], _PE_ROWS[n], orders[i])
        outs.append(out.reshape(_BATCH, _NUM_VARIATES, n, _D_MODEL))
    return tuple(outs)
